# Initial kernel scaffold; baseline (speedup 1.0000x reference)
import jax
import jax.numpy as jnp
from jax.experimental import pallas as pl


def kernel(x, y, rand):
    raise NotImplementedError("write your pallas kernel here")


# speedup vs baseline: 14.4645x; 14.4645x over previous
import functools

import jax
import jax.numpy as jnp
from jax import lax
from jax.experimental import pallas as pl
from jax.experimental.pallas import tpu as pltpu
from jax.experimental.pallas import tpu_sc as plsc

_PS = 7
_STRIDE = 2
_NPROJ = 256
_D = 147
_DP = 160
_L = 3721
_LP = 3840
_CHUNK = _LP // 16
_GCH = 64
_NCH = _LP // _GCH
_B = 4
_NTASK = _B * _NPROJ
_NTILE = 32
_TPT = _NTASK // _NTILE


def _patches_t(img):
    p = lax.conv_general_dilated_patches(
        img, filter_shape=(_PS, _PS), window_strides=(_STRIDE, _STRIDE),
        padding="VALID")
    return p.reshape(img.shape[0], _D, _L)


def _proj_tc_kernel(randT_ref, xT_ref, out_ref):
    r = randT_ref[...]
    mu = jnp.mean(r, axis=1, keepdims=True)
    var = jnp.sum((r - mu) ** 2, axis=1, keepdims=True) * (1.0 / (_D - 1))
    rn = r * lax.rsqrt(var)
    x = xT_ref[0]
    acc = lax.dot_general(rn, x, (((1,), (0,)), ((), ())),
                          preferred_element_type=jnp.float32)
    b = lax.bitcast_convert_type(acc, jnp.int32)
    mono = jnp.where(acc < 0, ~b, b | jnp.int32(-2147483648))
    col = lax.broadcasted_iota(jnp.int32, mono.shape, 1)
    out_ref[0] = jnp.where(col >= _L, jnp.int32(-1), mono)


_sc_mesh = plsc.VectorSubcoreMesh(core_axis_name="c", subcore_axis_name="s")


@functools.partial(
    pl.kernel,
    mesh=_sc_mesh,
    compiler_params=pltpu.CompilerParams(
        needs_layout_passes=False, use_tc_tiling_on_sc=False),
    out_type=jax.ShapeDtypeStruct((_NTASK * 16,), jnp.float32),
    scratch_types=[
        pltpu.VMEM((_LP,), jnp.int32),
        pltpu.VMEM((_LP,), jnp.int32),
        pltpu.VMEM((_LP,), jnp.int32),
        pltpu.VMEM((_LP,), jnp.int32),
        pltpu.VMEM((_LP,), jnp.int32),
        pltpu.VMEM((_LP,), jnp.int32),
        pltpu.VMEM((4096,), jnp.int32),
        pltpu.VMEM((_GCH, _DP), jnp.float32),
        pltpu.VMEM((_GCH, _DP), jnp.float32),
        pltpu.VMEM((_TPT * 16,), jnp.float32),
        pltpu.SemaphoreType.DMA,
        pltpu.SemaphoreType.DMA,
    ],
)
def _sc_swd_kernel(keys_hbm, xp_hbm, yp_hbm, out_hbm,
                   kA, kB, pA, pB, idxX, idxY, hist, bx, by, res,
                   semx, semy):
    wid = lax.axis_index("s") * 2 + lax.axis_index("c")
    lanes = lax.iota(jnp.int32, 16)
    ones = jnp.ones((16,), jnp.int32)
    zeros16 = jnp.zeros((16,), jnp.int32)
    gb0 = lanes * _CHUNK

    def radix_pass(kin, pin, kout, pout, shift, first, last, pbase):
        def zb(i, c):
            hist[pl.ds(i * 16, 16)] = zeros16
            return c
        lax.fori_loop(0, 256, zb, 0)

        def ph1(t, c):
            k = plsc.load_gather(kin, [gb0 + t])
            dg = lax.shift_right_logical(k, shift) & 255
            plsc.addupdate_scatter(hist, [dg * 16 + lanes], ones)
            return c
        lax.fori_loop(0, _CHUNK, ph1, 0)

        def ph2(i, carry):
            v = hist[pl.ds(i * 16, 16)]
            inc = plsc.cumsum(v)
            hist[pl.ds(i * 16, 16)] = inc - v + carry
            return carry + jnp.sum(v)
        lax.fori_loop(0, 256, ph2, jnp.int32(0))

        def ph3(t, c):
            gidx = gb0 + t
            k = plsc.load_gather(kin, [gidx])
            dg = lax.shift_right_logical(k, shift) & 255
            addr = dg * 16 + lanes
            off = plsc.load_gather(hist, [addr])
            if first:
                p = gidx + pbase
            else:
                p = plsc.load_gather(pin, [gidx])
            if not last:
                plsc.store_scatter(kout, [off], k)
            plsc.store_scatter(pout, [off], p)
            plsc.addupdate_scatter(hist, [addr], ones)
            return c
        lax.fori_loop(0, _CHUNK, ph3, 0)

    def sort_side(row, idx_out, pbase):
        pltpu.sync_copy(keys_hbm.at[row], kA)
        radix_pass(kA, None, kB, pB, 0, True, False, pbase)
        radix_pass(kB, pB, kA, pA, 8, False, False, pbase)
        radix_pass(kA, pA, kB, pB, 16, False, False, pbase)
        radix_pass(kB, pB, kA, idx_out, 24, False, True, pbase)

    def task_body(r, c):
        task = wid * _TPT + r
        s = task // _NPROJ
        j = task - s * _NPROJ
        pbase = s * _LP
        sort_side((s * 2) * _NPROJ + j, idxX, pbase)
        sort_side((s * 2 + 1) * _NPROJ + j, idxY, pbase)

        def chunk_body(ci, acc):
            cx = pltpu.async_copy(
                xp_hbm.at[idxX.at[pl.ds(ci * _GCH, _GCH)]], bx, semx)
            cy = pltpu.async_copy(
                yp_hbm.at[idxY.at[pl.ds(ci * _GCH, _GCH)]], by, semy)
            cx.wait()
            cy.wait()

            def rowloop(rr, a):
                for q in range(_DP // 16):
                    xv = bx[rr, pl.ds(q * 16, 16)]
                    yv = by[rr, pl.ds(q * 16, 16)]
                    a = a + jnp.abs(xv - yv)
                return a
            return lax.fori_loop(0, _GCH, rowloop, acc)

        acc = lax.fori_loop(0, _NCH, chunk_body, jnp.zeros((16,), jnp.float32))
        res[pl.ds(r * 16, 16)] = acc
        return c

    lax.fori_loop(0, _TPT, task_body, 0)
    pltpu.sync_copy(res, out_hbm.at[pl.ds(wid * _TPT * 16, _TPT * 16)])


def kernel(x, y, rand):
    xT = _patches_t(x)
    yT = _patches_t(y)
    xTp = jnp.pad(xT, ((0, 0), (0, 0), (0, _LP - _L)))
    yTp = jnp.pad(yT, ((0, 0), (0, 0), (0, _LP - _L)))
    xyT = jnp.stack([xTp, yTp], axis=1).reshape(2 * _B, _D, _LP)
    randT = jnp.transpose(rand)

    keys = pl.pallas_call(
        _proj_tc_kernel,
        grid=(2 * _B,),
        in_specs=[
            pl.BlockSpec((_NPROJ, _D), lambda i: (0, 0)),
            pl.BlockSpec((1, _D, _LP), lambda i: (i, 0, 0)),
        ],
        out_specs=pl.BlockSpec((1, _NPROJ, _LP), lambda i: (i, 0, 0)),
        out_shape=jax.ShapeDtypeStruct((2 * _B, _NPROJ, _LP), jnp.int32),
    )(randT, xyT)
    keys2 = keys.reshape(2 * _B * _NPROJ, _LP)

    xp = jnp.pad(jnp.transpose(xT, (0, 2, 1)),
                 ((0, 0), (0, _LP - _L), (0, _DP - _D))).reshape(_B * _LP, _DP)
    yp = jnp.pad(jnp.transpose(yT, (0, 2, 1)),
                 ((0, 0), (0, _LP - _L), (0, _DP - _D))).reshape(_B * _LP, _DP)

    sums = _sc_swd_kernel(keys2, xp, yp)
    per_sample = sums.reshape(_B, _NPROJ * 16).sum(axis=1)
    return jnp.mean(per_sample / jnp.float32(_L * _D * _NPROJ))


# speedup vs baseline: 22.4870x; 1.5546x over previous
import functools

import jax
import jax.numpy as jnp
from jax import lax
from jax.experimental import pallas as pl
from jax.experimental.pallas import tpu as pltpu
from jax.experimental.pallas import tpu_sc as plsc

_PS = 7
_STRIDE = 2
_NPROJ = 256
_D = 147
_DP = 160
_L = 3721
_LP = 3840
_CHUNK = _LP // 16
_GCH = 128
_NCH = _LP // _GCH
_B = 4
_NTASK = _B * _NPROJ
_NTILE = 32
_TPT = _NTASK // _NTILE


def _patches_t(img):
    p = lax.conv_general_dilated_patches(
        img, filter_shape=(_PS, _PS), window_strides=(_STRIDE, _STRIDE),
        padding="VALID")
    return p.reshape(img.shape[0], _D, _L)


def _proj_tc_kernel(randT_ref, xT_ref, out_ref):
    r = randT_ref[...]
    mu = jnp.mean(r, axis=1, keepdims=True)
    var = jnp.sum((r - mu) ** 2, axis=1, keepdims=True) * (1.0 / (_D - 1))
    rn = r * lax.rsqrt(var)
    x = xT_ref[0]
    acc = lax.dot_general(rn, x, (((1,), (0,)), ((), ())),
                          preferred_element_type=jnp.float32)
    b = lax.bitcast_convert_type(acc, jnp.int32)
    mono = jnp.where(acc < 0, ~b, b | jnp.int32(-2147483648))
    col = lax.broadcasted_iota(jnp.int32, mono.shape, 1)
    out_ref[0] = jnp.where(col >= _L, jnp.int32(-1), mono)


_sc_mesh = plsc.VectorSubcoreMesh(core_axis_name="c", subcore_axis_name="s")


@functools.partial(
    pl.kernel,
    mesh=_sc_mesh,
    compiler_params=pltpu.CompilerParams(
        needs_layout_passes=False, use_tc_tiling_on_sc=False),
    out_type=jax.ShapeDtypeStruct((_NTASK * 16,), jnp.float32),
    scratch_types=[
        pltpu.VMEM((_LP,), jnp.int32),
        pltpu.VMEM((_LP,), jnp.int32),
        pltpu.VMEM((_LP,), jnp.int32),
        pltpu.VMEM((_LP,), jnp.int32),
        pltpu.VMEM((_LP,), jnp.int32),
        pltpu.VMEM((_LP,), jnp.int32),
        pltpu.VMEM((4096,), jnp.int32),
        pltpu.VMEM((_GCH, _DP), jnp.bfloat16),
        pltpu.VMEM((_GCH, _DP), jnp.bfloat16),
        pltpu.VMEM((_GCH, _DP), jnp.bfloat16),
        pltpu.VMEM((_GCH, _DP), jnp.bfloat16),
        pltpu.VMEM((_TPT * 16,), jnp.float32),
        pltpu.SemaphoreType.DMA,
        pltpu.SemaphoreType.DMA,
        pltpu.SemaphoreType.DMA,
        pltpu.SemaphoreType.DMA,
    ],
)
def _sc_swd_kernel(keys_hbm, xp_hbm, yp_hbm, out_hbm,
                   kA, kB, pA, pB, idxX, idxY, hist,
                   bx0, bx1, by0, by1, res,
                   sx0, sx1, sy0, sy1):
    wid = lax.axis_index("s") * 2 + lax.axis_index("c")
    lanes = lax.iota(jnp.int32, 16)
    ones = jnp.ones((16,), jnp.int32)
    zeros16 = jnp.zeros((16,), jnp.int32)
    gb0 = lanes * _CHUNK

    def radix_pass(kin, pin, kout, pout, shift, first, last, pbase):
        def zb(i, c):
            hist[pl.ds(i * 16, 16)] = zeros16
            return c
        lax.fori_loop(0, 256, zb, 0)

        def ph1(t, c):
            k = plsc.load_gather(kin, [gb0 + t])
            dg = lax.shift_right_logical(k, shift) & 255
            plsc.addupdate_scatter(hist, [dg * 16 + lanes], ones)
            return c
        lax.fori_loop(0, _CHUNK, ph1, 0)

        def ph2(i, carry):
            v = hist[pl.ds(i * 16, 16)]
            inc = plsc.cumsum(v)
            hist[pl.ds(i * 16, 16)] = inc - v + carry
            return carry + jnp.sum(v)
        lax.fori_loop(0, 256, ph2, jnp.int32(0))

        def ph3(t, c):
            gidx = gb0 + t
            k = plsc.load_gather(kin, [gidx])
            dg = lax.shift_right_logical(k, shift) & 255
            addr = dg * 16 + lanes
            off = plsc.load_gather(hist, [addr])
            if first:
                p = gidx + pbase
            else:
                p = plsc.load_gather(pin, [gidx])
            if not last:
                plsc.store_scatter(kout, [off], k)
            plsc.store_scatter(pout, [off], p)
            plsc.addupdate_scatter(hist, [addr], ones)
            return c
        lax.fori_loop(0, _CHUNK, ph3, 0)

    def sort_side(row, idx_out, pbase):
        pltpu.sync_copy(keys_hbm.at[row], kA)
        radix_pass(kA, None, kB, pB, 0, True, False, pbase)
        radix_pass(kB, pB, kA, pA, 8, False, False, pbase)
        radix_pass(kA, pA, kB, pB, 16, False, False, pbase)
        radix_pass(kB, pB, kA, idx_out, 24, False, True, pbase)

    def task_body(r, c):
        task = wid * _TPT + r
        s = task // _NPROJ
        j = task - s * _NPROJ
        pbase = s * _LP
        sort_side((s * 2) * _NPROJ + j, idxX, pbase)
        sort_side((s * 2 + 1) * _NPROJ + j, idxY, pbase)

        bufs = ((bx0, by0, sx0, sy0), (bx1, by1, sx1, sy1))

        def issue(ci, slot):
            bx, by, sx, sy = bufs[slot]
            pltpu.async_copy(xp_hbm.at[idxX.at[pl.ds(ci * _GCH, _GCH)]],
                             bx, sx)
            pltpu.async_copy(yp_hbm.at[idxY.at[pl.ds(ci * _GCH, _GCH)]],
                             by, sy)

        issue(0, 0)
        issue(1, 1)

        def chunk_pair(i, acc):
            for slot in (0, 1):
                bx, by, sx, sy = bufs[slot]
                pltpu.make_async_copy(
                    xp_hbm.at[pl.ds(0, _GCH)], bx, sx).wait()
                pltpu.make_async_copy(
                    yp_hbm.at[pl.ds(0, _GCH)], by, sy).wait()

                def rowloop(rr, a):
                    for q in range(_DP // 32):
                        xv = bx[rr, pl.ds(q * 32, 32)]
                        yv = by[rr, pl.ds(q * 32, 32)]
                        d = jnp.abs(xv - yv)
                        lo, hi = plsc.unpack(
                            d, format=plsc.PackFormat.INTERLEAVED)
                        a = a + lo + hi
                    return a
                acc = lax.fori_loop(0, _GCH, rowloop, acc)

                @pl.when(2 * i + slot + 2 < _NCH)
                def _():
                    issue(2 * i + slot + 2, slot)
            return acc

        acc = lax.fori_loop(0, _NCH // 2, chunk_pair,
                            jnp.zeros((16,), jnp.float32))
        res[pl.ds(r * 16, 16)] = acc
        return c

    lax.fori_loop(0, _TPT, task_body, 0)
    pltpu.sync_copy(res, out_hbm.at[pl.ds(wid * _TPT * 16, _TPT * 16)])


def kernel(x, y, rand):
    xT = _patches_t(x)
    yT = _patches_t(y)
    xTp = jnp.pad(xT, ((0, 0), (0, 0), (0, _LP - _L)))
    yTp = jnp.pad(yT, ((0, 0), (0, 0), (0, _LP - _L)))
    xyT = jnp.stack([xTp, yTp], axis=1).reshape(2 * _B, _D, _LP)
    randT = jnp.transpose(rand)

    keys = pl.pallas_call(
        _proj_tc_kernel,
        grid=(2 * _B,),
        in_specs=[
            pl.BlockSpec((_NPROJ, _D), lambda i: (0, 0)),
            pl.BlockSpec((1, _D, _LP), lambda i: (i, 0, 0)),
        ],
        out_specs=pl.BlockSpec((1, _NPROJ, _LP), lambda i: (i, 0, 0)),
        out_shape=jax.ShapeDtypeStruct((2 * _B, _NPROJ, _LP), jnp.int32),
    )(randT, xyT)
    keys2 = keys.reshape(2 * _B * _NPROJ, _LP)

    xp = jnp.pad(jnp.transpose(xT, (0, 2, 1)),
                 ((0, 0), (0, _LP - _L), (0, _DP - _D))
                 ).reshape(_B * _LP, _DP).astype(jnp.bfloat16)
    yp = jnp.pad(jnp.transpose(yT, (0, 2, 1)),
                 ((0, 0), (0, _LP - _L), (0, _DP - _D))
                 ).reshape(_B * _LP, _DP).astype(jnp.bfloat16)

    sums = _sc_swd_kernel(keys2, xp, yp)
    per_sample = sums.reshape(_B, _NPROJ * 16).sum(axis=1)
    return jnp.mean(per_sample / jnp.float32(_L * _D * _NPROJ))
